# baseline (speedup 1.0000x reference)
import functools
import jax
import jax.numpy as jnp
from jax import lax
from jax.experimental import pallas as pl
from jax.experimental.pallas import tpu as pltpu, tpu_sc as plsc

_NC, _NS = 2, 16
_NW = _NC * _NS
_L = 16
_HALF = 32 * 1024
_SC_BATCH = 1
_TS = 512


def _sc_body(x_hbm, pe_hbm, out_hbm, pe_buf, xb0, xb1, l0, l1, s0, s1,
             *, n_batch):
    wid = lax.axis_index("s") * _NC + lax.axis_index("c")
    pe0 = wid * (2 * _HALF)
    xb = (xb0, xb1)
    lsem = (l0, l1)
    ssem = (s0, s1)
    pending_store = [None, None]

    def xoff(p, b):
        return b * (_NW * 2 * _HALF) + pe0 + p * _HALF

    def start_load(p, b, j):
        if pending_store[j] is not None:
            pending_store[j].wait()
            pending_store[j] = None
        return pltpu.async_copy(x_hbm.at[pl.ds(xoff(p, b), _HALF)], xb[j],
                                lsem[j])

    chunks = [(p, b) for p in range(2) for b in range(n_batch)]
    load = start_load(*chunks[0], 0)
    for i, (p, b) in enumerate(chunks):
        j = i % 2
        if b == 0:
            pltpu.sync_copy(pe_hbm.at[pl.ds(pe0 + p * _HALF, _HALF)], pe_buf)
        load.wait()
        if i + 1 < len(chunks):
            load = start_load(*chunks[i + 1], (i + 1) % 2)

        @plsc.parallel_loop(0, _HALF, step=_L, unroll=8)
        def _(k):
            plsc.addupdate(xb[j].at[pl.ds(k, _L)], pe_buf[pl.ds(k, _L)])

        pending_store[j] = pltpu.async_copy(
            xb[j], out_hbm.at[pl.ds(xoff(p, b), _HALF)], ssem[j])
    for j in range(2):
        if pending_store[j] is not None:
            pending_store[j].wait()


def _sc_add(x_flat, pe_flat, n_batch):
    mesh = plsc.VectorSubcoreMesh(core_axis_name="c", subcore_axis_name="s",
                                  num_cores=_NC, num_subcores=_NS)
    return pl.kernel(
        functools.partial(_sc_body, n_batch=n_batch),
        out_type=jax.ShapeDtypeStruct(x_flat.shape, jnp.float32),
        mesh=mesh,
        scratch_types=[
            pltpu.VMEM((_HALF,), jnp.float32),
            pltpu.VMEM((_HALF,), jnp.float32),
            pltpu.VMEM((_HALF,), jnp.float32),
            pltpu.SemaphoreType.DMA,
            pltpu.SemaphoreType.DMA,
            pltpu.SemaphoreType.DMA,
            pltpu.SemaphoreType.DMA,
        ],
    )(x_flat, pe_flat)


def _tc_body(x_ref, pe_ref, o_ref):
    o_ref[0] = x_ref[0] + pe_ref[...]


def _tc_add(x, pe_table):
    B, S, D = x.shape
    return pl.pallas_call(
        _tc_body,
        grid=(S // _TS, B),
        in_specs=[
            pl.BlockSpec((1, _TS, D), lambda i, b: (b, i, 0)),
            pl.BlockSpec((_TS, D), lambda i, b: (i, 0)),
        ],
        out_specs=pl.BlockSpec((1, _TS, D), lambda i, b: (b, i, 0)),
        out_shape=jax.ShapeDtypeStruct((B, S, D), x.dtype),
    )(x, pe_table)


def kernel(x, pe_table):
    B, S, D = x.shape
    k = _SC_BATCH
    pe_flat = pe_table.reshape(-1)
    out_sc = _sc_add(x[:k].reshape(-1), pe_flat, k).reshape(k, S, D)
    out_tc = _tc_add(x[k:], pe_table)
    return jnp.concatenate([out_sc, out_tc], axis=0)


# speedup vs baseline: 1.8783x; 1.8783x over previous
import functools
import jax
import jax.numpy as jnp
from jax import lax
from jax.experimental import pallas as pl
from jax.experimental.pallas import tpu as pltpu, tpu_sc as plsc

_NC, _NS = 2, 16
_NW = _NC * _NS
_L = 16
_CR = 32
_D = 1024


def _sc_body(x_hbm, pe_hbm, out_hbm, pe_buf, xb0, xb1, l0, l1, s0, s1,
             *, n_batch, seq):
    wid = lax.axis_index("s") * _NC + lax.axis_index("c")
    per0 = wid * (2 * _CR)
    xb = (xb0, xb1)
    lsem = (l0, l1)
    ssem = (s0, s1)
    pending_store = [None, None]

    def row0(p, b):
        return b * seq + per0 + p * _CR

    def start_load(p, b, j):
        if pending_store[j] is not None:
            pending_store[j].wait()
            pending_store[j] = None
        return pltpu.async_copy(x_hbm.at[pl.ds(row0(p, b), _CR), :], xb[j],
                                lsem[j])

    chunks = [(p, b) for p in range(2) for b in range(n_batch)]
    load = start_load(*chunks[0], 0)
    for i, (p, b) in enumerate(chunks):
        j = i % 2
        if b == 0:
            pltpu.sync_copy(pe_hbm.at[pl.ds(per0 + p * _CR, _CR), :], pe_buf)
        load.wait()
        if i + 1 < len(chunks):
            load = start_load(*chunks[i + 1], (i + 1) % 2)

        @plsc.parallel_loop(0, _CR, step=1, unroll=1)
        def _(r):
            for c in range(0, _D, _L):
                plsc.addupdate(xb[j].at[r, pl.ds(c, _L)],
                               pe_buf[r, pl.ds(c, _L)])

        pending_store[j] = pltpu.async_copy(
            xb[j], out_hbm.at[pl.ds(row0(p, b), _CR), :], ssem[j])
    for j in range(2):
        if pending_store[j] is not None:
            pending_store[j].wait()


def kernel(x, pe_table):
    B, S, D = x.shape
    mesh = plsc.VectorSubcoreMesh(core_axis_name="c", subcore_axis_name="s",
                                  num_cores=_NC, num_subcores=_NS)
    out = pl.kernel(
        functools.partial(_sc_body, n_batch=B, seq=S),
        out_type=jax.ShapeDtypeStruct((B * S, D), jnp.float32),
        mesh=mesh,
        scratch_types=[
            pltpu.VMEM((_CR, D), jnp.float32),
            pltpu.VMEM((_CR, D), jnp.float32),
            pltpu.VMEM((_CR, D), jnp.float32),
            pltpu.SemaphoreType.DMA,
            pltpu.SemaphoreType.DMA,
            pltpu.SemaphoreType.DMA,
            pltpu.SemaphoreType.DMA,
        ],
        compiler_params=pltpu.CompilerParams(use_tc_tiling_on_sc=True),
    )(x.reshape(B * S, D), pe_table)
    return out.reshape(B, S, D)
